# baseline (speedup 1.0000x reference)
import functools

import jax
import jax.numpy as jnp
from jax import lax
from jax.experimental import pallas as pl
from jax.experimental.pallas import tpu as pltpu
from jax.experimental.pallas import tpu_sc as plsc

VOCAB = 1000000
D_MODEL = 64
BATCH = 4096
HIST = 200

NUM_CORES = 2
NUM_SUBCORES = 16
NUM_WORKERS = NUM_CORES * NUM_SUBCORES

B_TOTAL = BATCH * HIST
PER_W = B_TOTAL // NUM_WORKERS
CHUNK = 800
NUM_CHUNKS = PER_W // CHUNK

_mesh = plsc.VectorSubcoreMesh(core_axis_name="c", subcore_axis_name="s")


@functools.partial(
    pl.kernel,
    out_type=jax.ShapeDtypeStruct((B_TOTAL, D_MODEL), jnp.float32),
    mesh=_mesh,
    scratch_types=[
        pltpu.VMEM((CHUNK,), jnp.int32),
        pltpu.VMEM((CHUNK, D_MODEL), jnp.float32),
        pltpu.SemaphoreType.DMA,
    ],
    compiler_params=pltpu.CompilerParams(use_tc_tiling_on_sc=False),
)
def _embed(idx_hbm, table_hbm, out_hbm, idx_v, rows_v, sem):
    wid = lax.axis_index("s") * NUM_CORES + lax.axis_index("c")
    base0 = wid * PER_W

    def step(c, carry):
        base = pl.multiple_of(base0 + c * CHUNK, 8)
        pltpu.sync_copy(idx_hbm.at[pl.ds(base, CHUNK)], idx_v)
        pltpu.async_copy(table_hbm.at[idx_v], rows_v, sem).wait()
        pltpu.sync_copy(rows_v, out_hbm.at[pl.ds(base, CHUNK)])
        return carry

    lax.fori_loop(0, NUM_CHUNKS, step, 0)


def kernel(X, table):
    idx = X.reshape(-1)
    out = _embed(idx, table)
    return out.reshape(X.shape + (table.shape[1],))


# speedup vs baseline: 1.0230x; 1.0230x over previous
import functools

import jax
import jax.numpy as jnp
from jax import lax
from jax.experimental import pallas as pl
from jax.experimental.pallas import tpu as pltpu
from jax.experimental.pallas import tpu_sc as plsc

VOCAB = 1000000
D_MODEL = 64
BATCH = 4096
HIST = 200

NUM_CORES = 2
NUM_SUBCORES = 16
NUM_WORKERS = NUM_CORES * NUM_SUBCORES

B_TOTAL = BATCH * HIST
PER_W = B_TOTAL // NUM_WORKERS
CHUNK = 400
NUM_CHUNKS = PER_W // CHUNK
NBUF = 4
LEAD = 2
NUM_GROUPS = NUM_CHUNKS // NBUF

_mesh = plsc.VectorSubcoreMesh(core_axis_name="c", subcore_axis_name="s")


@functools.partial(
    pl.kernel,
    out_type=jax.ShapeDtypeStruct((B_TOTAL, D_MODEL), jnp.float32),
    mesh=_mesh,
    scratch_types=[
        pltpu.VMEM((PER_W,), jnp.int32),
        pltpu.VMEM((NBUF, CHUNK, D_MODEL), jnp.float32),
        [pltpu.SemaphoreType.DMA] * NBUF,
        [pltpu.SemaphoreType.DMA] * NBUF,
    ],
    compiler_params=pltpu.CompilerParams(use_tc_tiling_on_sc=False),
)
def _embed(idx_hbm, table_hbm, out_hbm, idx_all, rows, gsems, osems):
    wid = lax.axis_index("s") * NUM_CORES + lax.axis_index("c")
    base0 = pl.multiple_of(wid * PER_W, 8)
    pltpu.sync_copy(idx_hbm.at[pl.ds(base0, PER_W)], idx_all)

    def gather_start(c, b):
        off = pl.multiple_of(c * CHUNK, 8)
        pltpu.async_copy(
            table_hbm.at[idx_all.at[pl.ds(off, CHUNK)]], rows.at[b], gsems[b]
        )

    def gather_wait(b):
        pltpu.make_async_copy(
            table_hbm.at[idx_all.at[pl.ds(0, CHUNK)]], rows.at[b], gsems[b]
        ).wait()

    def out_start(c, b):
        base = pl.multiple_of(base0 + c * CHUNK, 8)
        pltpu.async_copy(rows.at[b], out_hbm.at[pl.ds(base, CHUNK)], osems[b])

    def out_wait(b):
        pltpu.make_async_copy(
            rows.at[b], out_hbm.at[pl.ds(base0, CHUNK)], osems[b]
        ).wait()

    def body(g, carry):
        for b in range(NBUF):
            c = g * NBUF + b

            @pl.when(c < NUM_CHUNKS)
            def _():
                @pl.when(c >= NBUF)
                def _():
                    out_wait(b)

                gather_start(c, b)

            d = c - LEAD
            bd = (b - LEAD) % NBUF

            @pl.when((d >= 0) & (d < NUM_CHUNKS))
            def _():
                gather_wait(bd)
                out_start(d, bd)

        return carry

    lax.fori_loop(0, NUM_GROUPS + 1, body, 0)
    for b in range(NBUF):
        out_wait(b)


def kernel(X, table):
    idx = X.reshape(-1)
    out = _embed(idx, table)
    return out.reshape(X.shape + (table.shape[1],))
